# baseline (speedup 1.0000x reference)
import functools

import jax
import jax.numpy as jnp
from jax import lax
from jax.experimental import pallas as pl
from jax.experimental.pallas import tpu as pltpu
from jax.experimental.pallas import tpu_sc as plsc

N = 10000
E = 320000
D = 256
DQ = 64
NC = 2
NS = 16
EK = 80
NP = 10112

_MESH = dict(core_axis_name="c", subcore_axis_name="s")


def _sc_degree(src2):
    ones = jnp.ones((EK,), jnp.float32)
    zer = jnp.zeros((632,), jnp.float32)

    @functools.partial(
        pl.kernel,
        out_type=[jax.ShapeDtypeStruct((NP,), jnp.float32),
                  jax.ShapeDtypeStruct((NP,), jnp.float32)],
        mesh=plsc.VectorSubcoreMesh(**_MESH),
        scratch_types=[
            pltpu.VMEM((E // (NC * NS * EK), EK), jnp.int32),
            pltpu.VMEM((EK,), jnp.float32),
            pltpu.VMEM((632,), jnp.float32),
            pltpu.VMEM_SHARED((NP,), jnp.float32),
        ],
        compiler_params=pltpu.CompilerParams(use_tc_tiling_on_sc=False),
    )
    def deg_kernel(src_hbm, ones_hbm, zer_hbm, out0_hbm, out1_hbm,
                   idx_v, ones_v, stage_v, acc):
        c = lax.axis_index("c")
        s = lax.axis_index("s")

        pltpu.sync_copy(zer_hbm, stage_v)

        @pl.when(s < NS - 1)
        def _():
            pltpu.sync_copy(stage_v, acc.at[pl.ds(s * 632, 632)])

        @pl.when(s == NS - 1)
        def _():
            pltpu.sync_copy(stage_v.at[pl.ds(0, 520)],
                            acc.at[pl.ds((NS - 1) * 632, 520)])

        pltpu.sync_copy(ones_hbm, ones_v)

        nchunk = E // (NC * NS * EK)
        w = c * NS + s
        pltpu.sync_copy(src_hbm.at[pl.ds(w * nchunk, nchunk)], idx_v)
        plsc.subcore_barrier()

        def step(k, carry):
            pltpu.sync_copy(ones_v, acc.at[idx_v.at[k]], add=True)
            return carry

        lax.fori_loop(0, nchunk, step, 0)
        plsc.subcore_barrier()

        for ci, out_hbm in enumerate((out0_hbm, out1_hbm)):
            @pl.when((c == ci) & (s < NS - 1))
            def _(out_hbm=out_hbm):
                pltpu.sync_copy(acc.at[pl.ds(s * 632, 632)], stage_v)
                pltpu.sync_copy(stage_v, out_hbm.at[pl.ds(s * 632, 632)])

            @pl.when((c == ci) & (s == NS - 1))
            def _(out_hbm=out_hbm):
                pltpu.sync_copy(acc.at[pl.ds((NS - 1) * 632, 520)],
                                stage_v.at[pl.ds(0, 520)])
                pltpu.sync_copy(stage_v.at[pl.ds(0, 520)],
                                out_hbm.at[pl.ds((NS - 1) * 632, 520)])

    return deg_kernel(src2, ones, zer)


EKA = 128
EP = 321536
_NCH = EP // (NS * EKA)
NA = N + 8
_NB = 6
_AH = 3


def _sc_aggregate(hs_tab, src2e, spad, dst2e, dpad):
    zrows = jnp.zeros((640, DQ), jnp.float32)

    @functools.partial(
        pl.kernel,
        out_type=[jax.ShapeDtypeStruct((N, DQ), jnp.float32)] * 4,
        mesh=plsc.VectorSubcoreMesh(**_MESH),
        scratch_types=[
            pltpu.VMEM((_NCH, EKA), jnp.int32),
            pltpu.VMEM((_NCH, EKA), jnp.int32),
            [pltpu.VMEM((EKA, DQ), jnp.float32)] * _NB,
            pltpu.VMEM_SHARED((NA, DQ), jnp.float32),
            [pltpu.SemaphoreType.DMA] * _NB,
            [pltpu.SemaphoreType.DMA] * _NB,
        ],
        compiler_params=pltpu.CompilerParams(use_tc_tiling_on_sc=False),
    )
    def agg_kernel(tab_hbm, src_hbm, spad_hbm, dst_hbm, dpad_hbm, zr_hbm,
                   o0_hbm, o1_hbm, o2_hbm, o3_hbm,
                   idx_s, idx_d, bufs, acc, gsems, ssems):
        c = lax.axis_index("c")
        s = lax.axis_index("s")

        def zero_own_range():
            @pl.when(s < NS - 1)
            def _():
                pltpu.sync_copy(zr_hbm, acc.at[pl.ds(s * 640, 640)])

            @pl.when(s == NS - 1)
            def _():
                pltpu.sync_copy(zr_hbm.at[pl.ds(0, 400)],
                                acc.at[pl.ds(9600, 400)])

        def dump(out_hbm):
            @pl.when(s < NS - 1)
            def _():
                pltpu.sync_copy(acc.at[pl.ds(s * 640, 640)],
                                out_hbm.at[pl.ds(s * 640, 640)])

            @pl.when(s == NS - 1)
            def _():
                pltpu.sync_copy(acc.at[pl.ds(9600, 400)],
                                out_hbm.at[pl.ds(9600, 400)])

        @pl.when(s < NS - 1)
        def _():
            pltpu.sync_copy(src_hbm.at[pl.ds(s * _NCH, _NCH)], idx_s)
            pltpu.sync_copy(dst_hbm.at[pl.ds(s * _NCH, _NCH)], idx_d)

        @pl.when(s == NS - 1)
        def _():
            nreal = E // EKA - (NS - 1) * _NCH
            pltpu.sync_copy(src_hbm.at[pl.ds((NS - 1) * _NCH, nreal)],
                            idx_s.at[pl.ds(0, nreal)])
            pltpu.sync_copy(spad_hbm, idx_s.at[pl.ds(nreal, _NCH - nreal)])
            pltpu.sync_copy(dst_hbm.at[pl.ds((NS - 1) * _NCH, nreal)],
                            idx_d.at[pl.ds(0, nreal)])
            pltpu.sync_copy(dpad_hbm, idx_d.at[pl.ds(nreal, _NCH - nreal)])

        def edges(qi):
            tab_q = tab_hbm.at[pl.ds(qi * N, N)]

            def gissue(k, b):
                pltpu.async_copy(tab_q.at[idx_s.at[k]], bufs[b], gsems[b])

            def gwait(k, b):
                pltpu.make_async_copy(
                    tab_q.at[idx_s.at[k]], bufs[b], gsems[b]).wait()

            def sissue(k, b):
                pltpu.async_copy(bufs[b], acc.at[idx_d.at[k]], ssems[b],
                                 add=True)

            def swait(k, b):
                pltpu.make_async_copy(
                    bufs[b], acc.at[idx_d.at[k]], ssems[b]).wait()

            def step(k, jj, guard):
                gwait(k, jj)
                sissue(k, jj)
                kw = k - (_NB - _AH)
                ka = k + _AH
                bw = (jj - (_NB - _AH)) % _NB
                if guard:
                    @pl.when(kw >= 0)
                    def _():
                        swait(kw, bw)

                    @pl.when(ka < _NCH)
                    def _():
                        gissue(ka, bw)
                else:
                    if kw >= 0:
                        swait(kw, bw)
                    if ka < _NCH:
                        gissue(ka, bw)

            for k in range(_AH):
                gissue(k, k)

            def round_nb(m, carry):
                for j in range(_NB):
                    step(_NB * m + j, j, guard=True)
                return carry

            steady = _NCH // _NB
            lax.fori_loop(0, steady, round_nb, 0)
            for k in range(_NB * steady, _NCH):
                step(k, k % _NB, guard=False)
            for k in range(_NCH - (_NB - _AH), _NCH):
                swait(k, k % _NB)

        zero_own_range()
        plsc.subcore_barrier()

        outs = (o0_hbm, o1_hbm, o2_hbm, o3_hbm)
        for ci in range(NC):
            @pl.when(c == ci)
            def _(ci=ci):
                edges(2 * ci)
                plsc.subcore_barrier()
                dump(outs[2 * ci])
                zero_own_range()
                plsc.subcore_barrier()
                edges(2 * ci + 1)
                plsc.subcore_barrier()
                dump(outs[2 * ci + 1])

    return agg_kernel(hs_tab, src2e, spad, dst2e, dpad, zrows)


_BLK = 1000
_P = jax.lax.Precision.DEFAULT


def _dot(a, b):
    return jnp.dot(a, b, precision=_P, preferred_element_type=jnp.float32)


def _silu(v):
    return v * jax.nn.sigmoid(v)


def _quarters(hn, dinv):
    q = jnp.stack([hn[:, 0:64], hn[:, 64:128],
                   hn[:, 128:192], hn[:, 192:256]], axis=0)
    return q * dinv[None]


def _embed_kernel(x_ref, w_ref, b_ref, dg0_ref, dg1_ref,
                  h_ref, hs_ref, dinv_ref):
    h = _dot(x_ref[...], w_ref[...]) + b_ref[...]
    deg = dg0_ref[...] + dg1_ref[...]
    dinv = lax.rsqrt(deg)
    h_ref[...] = h
    hs_ref[...] = _quarters(h, dinv)
    dinv_ref[...] = dinv


def _tc_embed(x, emb_W, emb_b, dg0, dg1):
    grid = N // _BLK
    return pl.pallas_call(
        _embed_kernel,
        grid=(grid,),
        in_specs=[
            pl.BlockSpec((_BLK, 128), lambda i: (i, 0)),
            pl.BlockSpec((128, D), lambda i: (0, 0)),
            pl.BlockSpec((1, D), lambda i: (0, 0)),
            pl.BlockSpec((_BLK, 1), lambda i: (i, 0)),
            pl.BlockSpec((_BLK, 1), lambda i: (i, 0)),
        ],
        out_specs=[
            pl.BlockSpec((_BLK, D), lambda i: (i, 0)),
            pl.BlockSpec((4, _BLK, DQ), lambda i: (0, i, 0)),
            pl.BlockSpec((_BLK, 1), lambda i: (i, 0)),
        ],
        out_shape=[
            jax.ShapeDtypeStruct((N, D), jnp.float32),
            jax.ShapeDtypeStruct((4, N, DQ), jnp.float32),
            jax.ShapeDtypeStruct((N, 1), jnp.float32),
        ],
    )(x, emb_W, emb_b.reshape(1, D), dg0, dg1)


def _mlp_kernel(h_ref, a0_ref, a1_ref, a2_ref, a3_ref, dinv_ref,
                w1_ref, b1_ref, w2_ref, b2_ref, hn_ref, hs_ref):
    dinv = dinv_ref[...]
    h = h_ref[...]
    xc = jnp.concatenate(
        [h, a0_ref[...] * dinv, a1_ref[...] * dinv,
         a2_ref[...] * dinv, a3_ref[...] * dinv], axis=1)
    a = _silu(_dot(xc, w1_ref[...]) + b1_ref[...])
    hn = h + _dot(a, w2_ref[...]) + b2_ref[...]
    hn_ref[...] = hn
    hs_ref[...] = _quarters(hn, dinv)


def _final_kernel(h_ref, a0_ref, a1_ref, a2_ref, a3_ref, dinv_ref,
                  w1_ref, b1_ref, w2_ref, b2_ref,
                  pw1_ref, pb1_ref, pw2_ref, pb2_ref, out_ref):
    dinv = dinv_ref[...]
    h = h_ref[...]
    xc = jnp.concatenate(
        [h, a0_ref[...] * dinv, a1_ref[...] * dinv,
         a2_ref[...] * dinv, a3_ref[...] * dinv], axis=1)
    a = _silu(_dot(xc, w1_ref[...]) + b1_ref[...])
    hn = h + _dot(a, w2_ref[...]) + b2_ref[...]
    p = _dot(hn, pw1_ref[...]) + pb1_ref[...]
    out_ref[...] = _dot(p, pw2_ref[...]) + pb2_ref[...]


def _row_specs():
    return [
        pl.BlockSpec((_BLK, D), lambda i: (i, 0)),
        pl.BlockSpec((_BLK, DQ), lambda i: (i, 0)),
        pl.BlockSpec((_BLK, DQ), lambda i: (i, 0)),
        pl.BlockSpec((_BLK, DQ), lambda i: (i, 0)),
        pl.BlockSpec((_BLK, DQ), lambda i: (i, 0)),
        pl.BlockSpec((_BLK, 1), lambda i: (i, 0)),
    ]


def _w_specs():
    return [
        pl.BlockSpec((2 * D, D), lambda i: (0, 0)),
        pl.BlockSpec((1, D), lambda i: (0, 0)),
        pl.BlockSpec((D, D), lambda i: (0, 0)),
        pl.BlockSpec((1, D), lambda i: (0, 0)),
    ]


def _tc_mlp(h, aggs, dinv, lp):
    grid = N // _BLK
    return pl.pallas_call(
        _mlp_kernel,
        grid=(grid,),
        in_specs=_row_specs() + _w_specs(),
        out_specs=[
            pl.BlockSpec((_BLK, D), lambda i: (i, 0)),
            pl.BlockSpec((4, _BLK, DQ), lambda i: (0, i, 0)),
        ],
        out_shape=[
            jax.ShapeDtypeStruct((N, D), jnp.float32),
            jax.ShapeDtypeStruct((4, N, DQ), jnp.float32),
        ],
    )(h, *aggs, dinv, lp["nW1"], lp["nb1"].reshape(1, D),
      lp["nW2"], lp["nb2"].reshape(1, D))


def _tc_final(h, aggs, dinv, lp, params):
    grid = N // _BLK
    return pl.pallas_call(
        _final_kernel,
        grid=(grid,),
        in_specs=_row_specs() + _w_specs() + [
            pl.BlockSpec((D, D), lambda i: (0, 0)),
            pl.BlockSpec((1, D), lambda i: (0, 0)),
            pl.BlockSpec((D, 128), lambda i: (0, 0)),
            pl.BlockSpec((1, 128), lambda i: (0, 0)),
        ],
        out_specs=pl.BlockSpec((_BLK, 128), lambda i: (i, 0)),
        out_shape=jax.ShapeDtypeStruct((N, 128), jnp.float32),
    )(h, *aggs, dinv, lp["nW1"], lp["nb1"].reshape(1, D),
      lp["nW2"], lp["nb2"].reshape(1, D),
      params["pW1"], params["pb1"].reshape(1, D),
      params["pW2"], params["pb2"].reshape(1, 128))


def kernel(x, edge_index, edge_attr, batch, params):
    src = edge_index[0]
    dst = edge_index[1]
    pad = jnp.arange(EP - E, dtype=src.dtype)
    spad = (pad % N).reshape(-1, EKA)
    dpad = (N + pad % 8).reshape(-1, EKA)
    src2e = src.reshape(E // EKA, EKA)
    dst2e = dst.reshape(E // EKA, EKA)
    src2 = src.reshape(E // EK, EK)

    d0, d1 = _sc_degree(src2)
    h, hs, dinv = _tc_embed(x, params["emb_W"], params["emb_b"],
                            d0.reshape(NP, 1), d1.reshape(NP, 1))

    for li, lp in enumerate(params["layers"]):
        aggs = _sc_aggregate(hs.reshape(4 * N, DQ), src2e, spad, dst2e, dpad)
        if li < len(params["layers"]) - 1:
            h, hs = _tc_mlp(h, aggs, dinv, lp)
        else:
            out = _tc_final(h, aggs, dinv, lp, params)
    return out


# speedup vs baseline: 1.0026x; 1.0026x over previous
import functools

import jax
import jax.numpy as jnp
from jax import lax
from jax.experimental import pallas as pl
from jax.experimental.pallas import tpu as pltpu
from jax.experimental.pallas import tpu_sc as plsc

N = 10000
E = 320000
D = 256
DQ = 64
NC = 2
NS = 16
EK = 80
NP = 10112

_MESH = dict(core_axis_name="c", subcore_axis_name="s")


def _sc_degree(src2):
    ones = jnp.ones((EK,), jnp.float32)
    zer = jnp.zeros((632,), jnp.float32)

    @functools.partial(
        pl.kernel,
        out_type=[jax.ShapeDtypeStruct((NP,), jnp.float32),
                  jax.ShapeDtypeStruct((NP,), jnp.float32)],
        mesh=plsc.VectorSubcoreMesh(**_MESH),
        scratch_types=[
            pltpu.VMEM((E // (NC * NS * EK), EK), jnp.int32),
            pltpu.VMEM((EK,), jnp.float32),
            pltpu.VMEM((632,), jnp.float32),
            pltpu.VMEM_SHARED((NP,), jnp.float32),
        ],
        compiler_params=pltpu.CompilerParams(use_tc_tiling_on_sc=False),
    )
    def deg_kernel(src_hbm, ones_hbm, zer_hbm, out0_hbm, out1_hbm,
                   idx_v, ones_v, stage_v, acc):
        c = lax.axis_index("c")
        s = lax.axis_index("s")

        pltpu.sync_copy(zer_hbm, stage_v)

        @pl.when(s < NS - 1)
        def _():
            pltpu.sync_copy(stage_v, acc.at[pl.ds(s * 632, 632)])

        @pl.when(s == NS - 1)
        def _():
            pltpu.sync_copy(stage_v.at[pl.ds(0, 520)],
                            acc.at[pl.ds((NS - 1) * 632, 520)])

        pltpu.sync_copy(ones_hbm, ones_v)

        nchunk = E // (NC * NS * EK)
        w = c * NS + s
        pltpu.sync_copy(src_hbm.at[pl.ds(w * nchunk, nchunk)], idx_v)
        plsc.subcore_barrier()

        def step(k, carry):
            pltpu.sync_copy(ones_v, acc.at[idx_v.at[k]], add=True)
            return carry

        lax.fori_loop(0, nchunk, step, 0)
        plsc.subcore_barrier()

        for ci, out_hbm in enumerate((out0_hbm, out1_hbm)):
            @pl.when((c == ci) & (s < NS - 1))
            def _(out_hbm=out_hbm):
                pltpu.sync_copy(acc.at[pl.ds(s * 632, 632)], stage_v)
                pltpu.sync_copy(stage_v, out_hbm.at[pl.ds(s * 632, 632)])

            @pl.when((c == ci) & (s == NS - 1))
            def _(out_hbm=out_hbm):
                pltpu.sync_copy(acc.at[pl.ds((NS - 1) * 632, 520)],
                                stage_v.at[pl.ds(0, 520)])
                pltpu.sync_copy(stage_v.at[pl.ds(0, 520)],
                                out_hbm.at[pl.ds((NS - 1) * 632, 520)])

    return deg_kernel(src2, ones, zer)


EKA = 128
EP = 321536
_NCH = EP // (NS * EKA)
NA = N + 8
_NB = 6
_AH = 3


def _sc_aggregate(hs_tab, src2e, spad, dst2e, dpad):
    zrows = jnp.zeros((128, DQ), jnp.float32)

    @functools.partial(
        pl.kernel,
        out_type=[jax.ShapeDtypeStruct((N, DQ), jnp.float32)] * 4,
        mesh=plsc.VectorSubcoreMesh(**_MESH),
        scratch_types=[
            pltpu.VMEM((_NCH, EKA), jnp.int32),
            pltpu.VMEM((_NCH, EKA), jnp.int32),
            [pltpu.VMEM((EKA, DQ), jnp.float32)] * _NB,
            pltpu.VMEM_SHARED((NA, DQ), jnp.float32),
            [pltpu.SemaphoreType.DMA] * _NB,
            [pltpu.SemaphoreType.DMA] * _NB,
        ],
        compiler_params=pltpu.CompilerParams(use_tc_tiling_on_sc=False),
    )
    def agg_kernel(tab_hbm, src_hbm, spad_hbm, dst_hbm, dpad_hbm, zr_hbm,
                   o0_hbm, o1_hbm, o2_hbm, o3_hbm,
                   idx_s, idx_d, bufs, acc, gsems, ssems):
        c = lax.axis_index("c")
        s = lax.axis_index("s")

        def zero_own_range():
            zb = bufs[_NB - 1]
            pltpu.sync_copy(zr_hbm, zb)

            @pl.when(s < NS - 1)
            def _():
                def zs(j, carry):
                    pltpu.sync_copy(zb, acc.at[pl.ds(s * 640 + j * 128, 128)])
                    return carry
                lax.fori_loop(0, 5, zs, 0)

            @pl.when(s == NS - 1)
            def _():
                def zs(j, carry):
                    pltpu.sync_copy(zb.at[pl.ds(0, 80)],
                                    acc.at[pl.ds(9600 + j * 80, 80)])
                    return carry
                lax.fori_loop(0, 5, zs, 0)

        def dump(out_hbm):
            def go(nrows, base):
                def rng(j):
                    return pl.ds(base + j * nrows, nrows)

                for j in range(5):
                    t = j % 2
                    bv = bufs[t].at[pl.ds(0, nrows)]
                    if j >= 2:
                        pltpu.make_async_copy(
                            bv, out_hbm.at[rng(j - 2)], gsems[t]).wait()
                    pltpu.sync_copy(acc.at[rng(j)], bv)
                    pltpu.async_copy(bv, out_hbm.at[rng(j)], gsems[t])
                for j in (3, 4):
                    t = j % 2
                    pltpu.make_async_copy(
                        bufs[t].at[pl.ds(0, nrows)],
                        out_hbm.at[rng(j)], gsems[t]).wait()

            @pl.when(s < NS - 1)
            def _():
                go(128, s * 640)

            @pl.when(s == NS - 1)
            def _():
                go(80, 9600)

        @pl.when(s < NS - 1)
        def _():
            pltpu.sync_copy(src_hbm.at[pl.ds(s * _NCH, _NCH)], idx_s)
            pltpu.sync_copy(dst_hbm.at[pl.ds(s * _NCH, _NCH)], idx_d)

        @pl.when(s == NS - 1)
        def _():
            nreal = E // EKA - (NS - 1) * _NCH
            pltpu.sync_copy(src_hbm.at[pl.ds((NS - 1) * _NCH, nreal)],
                            idx_s.at[pl.ds(0, nreal)])
            pltpu.sync_copy(spad_hbm, idx_s.at[pl.ds(nreal, _NCH - nreal)])
            pltpu.sync_copy(dst_hbm.at[pl.ds((NS - 1) * _NCH, nreal)],
                            idx_d.at[pl.ds(0, nreal)])
            pltpu.sync_copy(dpad_hbm, idx_d.at[pl.ds(nreal, _NCH - nreal)])

        def edges(qi):
            tab_q = tab_hbm.at[pl.ds(qi * N, N)]

            def gissue(k, b):
                pltpu.async_copy(tab_q.at[idx_s.at[k]], bufs[b], gsems[b])

            def gwait(k, b):
                pltpu.make_async_copy(
                    tab_q.at[idx_s.at[k]], bufs[b], gsems[b]).wait()

            def sissue(k, b):
                pltpu.async_copy(bufs[b], acc.at[idx_d.at[k]], ssems[b],
                                 add=True)

            def swait(k, b):
                pltpu.make_async_copy(
                    bufs[b], acc.at[idx_d.at[k]], ssems[b]).wait()

            def step(k, jj, guard):
                gwait(k, jj)
                sissue(k, jj)
                kw = k - (_NB - _AH)
                ka = k + _AH
                bw = (jj - (_NB - _AH)) % _NB
                if guard:
                    @pl.when(kw >= 0)
                    def _():
                        swait(kw, bw)

                    @pl.when(ka < _NCH)
                    def _():
                        gissue(ka, bw)
                else:
                    if kw >= 0:
                        swait(kw, bw)
                    if ka < _NCH:
                        gissue(ka, bw)

            for k in range(_AH):
                gissue(k, k)

            def round_nb(m, carry):
                for j in range(_NB):
                    step(_NB * m + j, j, guard=True)
                return carry

            steady = _NCH // _NB
            lax.fori_loop(0, steady, round_nb, 0)
            for k in range(_NB * steady, _NCH):
                step(k, k % _NB, guard=False)
            for k in range(_NCH - (_NB - _AH), _NCH):
                swait(k, k % _NB)

        zero_own_range()
        plsc.subcore_barrier()

        outs = (o0_hbm, o1_hbm, o2_hbm, o3_hbm)
        for ci in range(NC):
            @pl.when(c == ci)
            def _(ci=ci):
                edges(2 * ci)
                plsc.subcore_barrier()
                dump(outs[2 * ci])
                zero_own_range()
                plsc.subcore_barrier()
                edges(2 * ci + 1)
                plsc.subcore_barrier()
                dump(outs[2 * ci + 1])

    return agg_kernel(hs_tab, src2e, spad, dst2e, dpad, zrows)


_BLK = 1000
_P = jax.lax.Precision.DEFAULT


def _dot(a, b):
    return jnp.dot(a, b, precision=_P, preferred_element_type=jnp.float32)


def _silu(v):
    return v * jax.nn.sigmoid(v)


def _quarters(hn, dinv):
    q = jnp.stack([hn[:, 0:64], hn[:, 64:128],
                   hn[:, 128:192], hn[:, 192:256]], axis=0)
    return q * dinv[None]


def _embed_kernel(x_ref, w_ref, b_ref, dg0_ref, dg1_ref,
                  h_ref, hs_ref, dinv_ref):
    h = _dot(x_ref[...], w_ref[...]) + b_ref[...]
    deg = dg0_ref[...] + dg1_ref[...]
    dinv = lax.rsqrt(deg)
    h_ref[...] = h
    hs_ref[...] = _quarters(h, dinv)
    dinv_ref[...] = dinv


def _tc_embed(x, emb_W, emb_b, dg0, dg1):
    grid = N // _BLK
    return pl.pallas_call(
        _embed_kernel,
        grid=(grid,),
        in_specs=[
            pl.BlockSpec((_BLK, 128), lambda i: (i, 0)),
            pl.BlockSpec((128, D), lambda i: (0, 0)),
            pl.BlockSpec((1, D), lambda i: (0, 0)),
            pl.BlockSpec((_BLK, 1), lambda i: (i, 0)),
            pl.BlockSpec((_BLK, 1), lambda i: (i, 0)),
        ],
        out_specs=[
            pl.BlockSpec((_BLK, D), lambda i: (i, 0)),
            pl.BlockSpec((4, _BLK, DQ), lambda i: (0, i, 0)),
            pl.BlockSpec((_BLK, 1), lambda i: (i, 0)),
        ],
        out_shape=[
            jax.ShapeDtypeStruct((N, D), jnp.float32),
            jax.ShapeDtypeStruct((4, N, DQ), jnp.float32),
            jax.ShapeDtypeStruct((N, 1), jnp.float32),
        ],
    )(x, emb_W, emb_b.reshape(1, D), dg0, dg1)


def _mlp_kernel(h_ref, a0_ref, a1_ref, a2_ref, a3_ref, dinv_ref,
                w1_ref, b1_ref, w2_ref, b2_ref, hn_ref, hs_ref):
    dinv = dinv_ref[...]
    h = h_ref[...]
    xc = jnp.concatenate(
        [h, a0_ref[...] * dinv, a1_ref[...] * dinv,
         a2_ref[...] * dinv, a3_ref[...] * dinv], axis=1)
    a = _silu(_dot(xc, w1_ref[...]) + b1_ref[...])
    hn = h + _dot(a, w2_ref[...]) + b2_ref[...]
    hn_ref[...] = hn
    hs_ref[...] = _quarters(hn, dinv)


def _final_kernel(h_ref, a0_ref, a1_ref, a2_ref, a3_ref, dinv_ref,
                  w1_ref, b1_ref, w2_ref, b2_ref,
                  pw1_ref, pb1_ref, pw2_ref, pb2_ref, out_ref):
    dinv = dinv_ref[...]
    h = h_ref[...]
    xc = jnp.concatenate(
        [h, a0_ref[...] * dinv, a1_ref[...] * dinv,
         a2_ref[...] * dinv, a3_ref[...] * dinv], axis=1)
    a = _silu(_dot(xc, w1_ref[...]) + b1_ref[...])
    hn = h + _dot(a, w2_ref[...]) + b2_ref[...]
    p = _dot(hn, pw1_ref[...]) + pb1_ref[...]
    out_ref[...] = _dot(p, pw2_ref[...]) + pb2_ref[...]


def _row_specs():
    return [
        pl.BlockSpec((_BLK, D), lambda i: (i, 0)),
        pl.BlockSpec((_BLK, DQ), lambda i: (i, 0)),
        pl.BlockSpec((_BLK, DQ), lambda i: (i, 0)),
        pl.BlockSpec((_BLK, DQ), lambda i: (i, 0)),
        pl.BlockSpec((_BLK, DQ), lambda i: (i, 0)),
        pl.BlockSpec((_BLK, 1), lambda i: (i, 0)),
    ]


def _w_specs():
    return [
        pl.BlockSpec((2 * D, D), lambda i: (0, 0)),
        pl.BlockSpec((1, D), lambda i: (0, 0)),
        pl.BlockSpec((D, D), lambda i: (0, 0)),
        pl.BlockSpec((1, D), lambda i: (0, 0)),
    ]


def _tc_mlp(h, aggs, dinv, lp):
    grid = N // _BLK
    return pl.pallas_call(
        _mlp_kernel,
        grid=(grid,),
        in_specs=_row_specs() + _w_specs(),
        out_specs=[
            pl.BlockSpec((_BLK, D), lambda i: (i, 0)),
            pl.BlockSpec((4, _BLK, DQ), lambda i: (0, i, 0)),
        ],
        out_shape=[
            jax.ShapeDtypeStruct((N, D), jnp.float32),
            jax.ShapeDtypeStruct((4, N, DQ), jnp.float32),
        ],
    )(h, *aggs, dinv, lp["nW1"], lp["nb1"].reshape(1, D),
      lp["nW2"], lp["nb2"].reshape(1, D))


def _tc_final(h, aggs, dinv, lp, params):
    grid = N // _BLK
    return pl.pallas_call(
        _final_kernel,
        grid=(grid,),
        in_specs=_row_specs() + _w_specs() + [
            pl.BlockSpec((D, D), lambda i: (0, 0)),
            pl.BlockSpec((1, D), lambda i: (0, 0)),
            pl.BlockSpec((D, 128), lambda i: (0, 0)),
            pl.BlockSpec((1, 128), lambda i: (0, 0)),
        ],
        out_specs=pl.BlockSpec((_BLK, 128), lambda i: (i, 0)),
        out_shape=jax.ShapeDtypeStruct((N, 128), jnp.float32),
    )(h, *aggs, dinv, lp["nW1"], lp["nb1"].reshape(1, D),
      lp["nW2"], lp["nb2"].reshape(1, D),
      params["pW1"], params["pb1"].reshape(1, D),
      params["pW2"], params["pb2"].reshape(1, 128))


def kernel(x, edge_index, edge_attr, batch, params):
    src = edge_index[0]
    dst = edge_index[1]
    pad = jnp.arange(EP - E, dtype=src.dtype)
    spad = (pad % N).reshape(-1, EKA)
    dpad = (N + pad % 8).reshape(-1, EKA)
    src2e = src.reshape(E // EKA, EKA)
    dst2e = dst.reshape(E // EKA, EKA)
    src2 = src.reshape(E // EK, EK)

    d0, d1 = _sc_degree(src2)
    h, hs, dinv = _tc_embed(x, params["emb_W"], params["emb_b"],
                            d0.reshape(NP, 1), d1.reshape(NP, 1))

    for li, lp in enumerate(params["layers"]):
        aggs = _sc_aggregate(hs.reshape(4 * N, DQ), src2e, spad, dst2e, dpad)
        if li < len(params["layers"]) - 1:
            h, hs = _tc_mlp(h, aggs, dinv, lp)
        else:
            out = _tc_final(h, aggs, dinv, lp, params)
    return out


# speedup vs baseline: 1.0716x; 1.0688x over previous
import functools

import jax
import jax.numpy as jnp
from jax import lax
from jax.experimental import pallas as pl
from jax.experimental.pallas import tpu as pltpu
from jax.experimental.pallas import tpu_sc as plsc

N = 10000
E = 320000
D = 256
DQ = 64
NC = 2
NS = 16
EK = 80
NP = 10112

_MESH = dict(core_axis_name="c", subcore_axis_name="s")


def _sc_degree(src2):
    ones = jnp.ones((EK,), jnp.float32)
    zer = jnp.zeros((632,), jnp.float32)

    @functools.partial(
        pl.kernel,
        out_type=[jax.ShapeDtypeStruct((NP,), jnp.float32),
                  jax.ShapeDtypeStruct((NP,), jnp.float32)],
        mesh=plsc.VectorSubcoreMesh(**_MESH),
        scratch_types=[
            pltpu.VMEM((E // (NC * NS * EK), EK), jnp.int32),
            pltpu.VMEM((EK,), jnp.float32),
            pltpu.VMEM((632,), jnp.float32),
            pltpu.VMEM_SHARED((NP,), jnp.float32),
        ],
        compiler_params=pltpu.CompilerParams(use_tc_tiling_on_sc=False),
    )
    def deg_kernel(src_hbm, ones_hbm, zer_hbm, out0_hbm, out1_hbm,
                   idx_v, ones_v, stage_v, acc):
        c = lax.axis_index("c")
        s = lax.axis_index("s")

        pltpu.sync_copy(zer_hbm, stage_v)

        @pl.when(s < NS - 1)
        def _():
            pltpu.sync_copy(stage_v, acc.at[pl.ds(s * 632, 632)])

        @pl.when(s == NS - 1)
        def _():
            pltpu.sync_copy(stage_v.at[pl.ds(0, 520)],
                            acc.at[pl.ds((NS - 1) * 632, 520)])

        pltpu.sync_copy(ones_hbm, ones_v)

        nchunk = E // (NC * NS * EK)
        w = c * NS + s
        pltpu.sync_copy(src_hbm.at[pl.ds(w * nchunk, nchunk)], idx_v)
        plsc.subcore_barrier()

        def step(k, carry):
            pltpu.sync_copy(ones_v, acc.at[idx_v.at[k]], add=True)
            return carry

        lax.fori_loop(0, nchunk, step, 0)
        plsc.subcore_barrier()

        for ci, out_hbm in enumerate((out0_hbm, out1_hbm)):
            @pl.when((c == ci) & (s < NS - 1))
            def _(out_hbm=out_hbm):
                pltpu.sync_copy(acc.at[pl.ds(s * 632, 632)], stage_v)
                pltpu.sync_copy(stage_v, out_hbm.at[pl.ds(s * 632, 632)])

            @pl.when((c == ci) & (s == NS - 1))
            def _(out_hbm=out_hbm):
                pltpu.sync_copy(acc.at[pl.ds((NS - 1) * 632, 520)],
                                stage_v.at[pl.ds(0, 520)])
                pltpu.sync_copy(stage_v.at[pl.ds(0, 520)],
                                out_hbm.at[pl.ds((NS - 1) * 632, 520)])

    return deg_kernel(src2, ones, zer)


EKA = 128
EP = 321536
_NCH = EP // (NS * EKA)
NA = N + 8
_NB = 5
_AH = 3


def _sc_aggregate(hs_tab, src2e, spad, dst2e, dpad):
    zrows = jnp.zeros((128, DQ), jnp.float32)

    @functools.partial(
        pl.kernel,
        out_type=[jax.ShapeDtypeStruct((N, DQ), jnp.float32)] * 4,
        mesh=plsc.VectorSubcoreMesh(**_MESH),
        scratch_types=[
            pltpu.VMEM((_NCH, EKA), jnp.int32),
            pltpu.VMEM((_NCH, EKA), jnp.int32),
            [pltpu.VMEM((EKA, DQ), jnp.float32)] * _NB,
            pltpu.VMEM_SHARED((NA, DQ), jnp.float32),
            [pltpu.SemaphoreType.DMA] * _NB,
            [pltpu.SemaphoreType.DMA] * _NB,
        ],
        compiler_params=pltpu.CompilerParams(use_tc_tiling_on_sc=False),
    )
    def agg_kernel(tab_hbm, src_hbm, spad_hbm, dst_hbm, dpad_hbm, zr_hbm,
                   o0_hbm, o1_hbm, o2_hbm, o3_hbm,
                   idx_s, idx_d, bufs, acc, gsems, ssems):
        c = lax.axis_index("c")
        s = lax.axis_index("s")

        def zero_own_range():
            zb = bufs[_NB - 1]
            pltpu.sync_copy(zr_hbm, zb)

            @pl.when(s < NS - 1)
            def _():
                def zs(j, carry):
                    pltpu.sync_copy(zb, acc.at[pl.ds(s * 640 + j * 128, 128)])
                    return carry
                lax.fori_loop(0, 5, zs, 0)

            @pl.when(s == NS - 1)
            def _():
                def zs(j, carry):
                    pltpu.sync_copy(zb.at[pl.ds(0, 80)],
                                    acc.at[pl.ds(9600 + j * 80, 80)])
                    return carry
                lax.fori_loop(0, 5, zs, 0)

        def dump(out_hbm):
            def go(nrows, base):
                def rng(j):
                    return pl.ds(base + j * nrows, nrows)

                for j in range(5):
                    t = j % 2
                    bv = bufs[t].at[pl.ds(0, nrows)]
                    if j >= 2:
                        pltpu.make_async_copy(
                            bv, out_hbm.at[rng(j - 2)], gsems[t]).wait()
                    pltpu.sync_copy(acc.at[rng(j)], bv)
                    pltpu.async_copy(bv, out_hbm.at[rng(j)], gsems[t])
                for j in (3, 4):
                    t = j % 2
                    pltpu.make_async_copy(
                        bufs[t].at[pl.ds(0, nrows)],
                        out_hbm.at[rng(j)], gsems[t]).wait()

            @pl.when(s < NS - 1)
            def _():
                go(128, s * 640)

            @pl.when(s == NS - 1)
            def _():
                go(80, 9600)

        @pl.when(s < NS - 1)
        def _():
            pltpu.sync_copy(src_hbm.at[pl.ds(s * _NCH, _NCH)], idx_s)
            pltpu.sync_copy(dst_hbm.at[pl.ds(s * _NCH, _NCH)], idx_d)

        @pl.when(s == NS - 1)
        def _():
            nreal = E // EKA - (NS - 1) * _NCH
            pltpu.sync_copy(src_hbm.at[pl.ds((NS - 1) * _NCH, nreal)],
                            idx_s.at[pl.ds(0, nreal)])
            pltpu.sync_copy(spad_hbm, idx_s.at[pl.ds(nreal, _NCH - nreal)])
            pltpu.sync_copy(dst_hbm.at[pl.ds((NS - 1) * _NCH, nreal)],
                            idx_d.at[pl.ds(0, nreal)])
            pltpu.sync_copy(dpad_hbm, idx_d.at[pl.ds(nreal, _NCH - nreal)])

        def edges(qi):
            tab_q = tab_hbm.at[pl.ds(qi * N, N)]

            def gissue(k, b):
                pltpu.async_copy(tab_q.at[idx_s.at[k]], bufs[b], gsems[b])

            def gwait(k, b):
                pltpu.make_async_copy(
                    tab_q.at[idx_s.at[k]], bufs[b], gsems[b]).wait()

            def sissue(k, b):
                pltpu.async_copy(bufs[b], acc.at[idx_d.at[k]], ssems[b],
                                 add=True)

            def swait(k, b):
                pltpu.make_async_copy(
                    bufs[b], acc.at[idx_d.at[k]], ssems[b]).wait()

            def step(k, jj, guard):
                gwait(k, jj)
                sissue(k, jj)
                kw = k - (_NB - _AH)
                ka = k + _AH
                bw = (jj - (_NB - _AH)) % _NB
                if guard:
                    @pl.when(kw >= 0)
                    def _():
                        swait(kw, bw)

                    @pl.when(ka < _NCH)
                    def _():
                        gissue(ka, bw)
                else:
                    if kw >= 0:
                        swait(kw, bw)
                    if ka < _NCH:
                        gissue(ka, bw)

            for k in range(_AH):
                gissue(k, k)

            def round_nb(m, carry):
                for j in range(_NB):
                    step(_NB * m + j, j, guard=True)
                return carry

            steady = _NCH // _NB
            lax.fori_loop(0, steady, round_nb, 0)
            for k in range(_NB * steady, _NCH):
                step(k, k % _NB, guard=False)
            for k in range(_NCH - (_NB - _AH), _NCH):
                swait(k, k % _NB)

        zero_own_range()
        plsc.subcore_barrier()

        outs = (o0_hbm, o1_hbm, o2_hbm, o3_hbm)
        for ci in range(NC):
            @pl.when(c == ci)
            def _(ci=ci):
                edges(2 * ci)
                plsc.subcore_barrier()
                dump(outs[2 * ci])
                zero_own_range()
                plsc.subcore_barrier()
                edges(2 * ci + 1)
                plsc.subcore_barrier()
                dump(outs[2 * ci + 1])

    return agg_kernel(hs_tab, src2e, spad, dst2e, dpad, zrows)


_BLK = 1000
_P = jax.lax.Precision.DEFAULT


def _dot(a, b):
    return jnp.dot(a, b, precision=_P, preferred_element_type=jnp.float32)


def _silu(v):
    return v * jax.nn.sigmoid(v)


def _quarters(hn, dinv):
    q = jnp.stack([hn[:, 0:64], hn[:, 64:128],
                   hn[:, 128:192], hn[:, 192:256]], axis=0)
    return q * dinv[None]


def _embed_kernel(x_ref, w_ref, b_ref, dg0_ref, dg1_ref,
                  h_ref, hs_ref, dinv_ref):
    h = _dot(x_ref[...], w_ref[...]) + b_ref[...]
    deg = dg0_ref[...] + dg1_ref[...]
    dinv = lax.rsqrt(deg)
    h_ref[...] = h
    hs_ref[...] = _quarters(h, dinv)
    dinv_ref[...] = dinv


def _tc_embed(x, emb_W, emb_b, dg0, dg1):
    grid = N // _BLK
    return pl.pallas_call(
        _embed_kernel,
        grid=(grid,),
        in_specs=[
            pl.BlockSpec((_BLK, 128), lambda i: (i, 0)),
            pl.BlockSpec((128, D), lambda i: (0, 0)),
            pl.BlockSpec((1, D), lambda i: (0, 0)),
            pl.BlockSpec((_BLK, 1), lambda i: (i, 0)),
            pl.BlockSpec((_BLK, 1), lambda i: (i, 0)),
        ],
        out_specs=[
            pl.BlockSpec((_BLK, D), lambda i: (i, 0)),
            pl.BlockSpec((4, _BLK, DQ), lambda i: (0, i, 0)),
            pl.BlockSpec((_BLK, 1), lambda i: (i, 0)),
        ],
        out_shape=[
            jax.ShapeDtypeStruct((N, D), jnp.float32),
            jax.ShapeDtypeStruct((4, N, DQ), jnp.float32),
            jax.ShapeDtypeStruct((N, 1), jnp.float32),
        ],
    )(x, emb_W, emb_b.reshape(1, D), dg0, dg1)


def _mlp_kernel(h_ref, a0_ref, a1_ref, a2_ref, a3_ref, dinv_ref,
                w1_ref, b1_ref, w2_ref, b2_ref, hn_ref, hs_ref):
    dinv = dinv_ref[...]
    h = h_ref[...]
    xc = jnp.concatenate(
        [h, a0_ref[...] * dinv, a1_ref[...] * dinv,
         a2_ref[...] * dinv, a3_ref[...] * dinv], axis=1)
    a = _silu(_dot(xc, w1_ref[...]) + b1_ref[...])
    hn = h + _dot(a, w2_ref[...]) + b2_ref[...]
    hn_ref[...] = hn
    hs_ref[...] = _quarters(hn, dinv)


def _final_kernel(h_ref, a0_ref, a1_ref, a2_ref, a3_ref, dinv_ref,
                  w1_ref, b1_ref, w2_ref, b2_ref,
                  pw1_ref, pb1_ref, pw2_ref, pb2_ref, out_ref):
    dinv = dinv_ref[...]
    h = h_ref[...]
    xc = jnp.concatenate(
        [h, a0_ref[...] * dinv, a1_ref[...] * dinv,
         a2_ref[...] * dinv, a3_ref[...] * dinv], axis=1)
    a = _silu(_dot(xc, w1_ref[...]) + b1_ref[...])
    hn = h + _dot(a, w2_ref[...]) + b2_ref[...]
    p = _dot(hn, pw1_ref[...]) + pb1_ref[...]
    out_ref[...] = _dot(p, pw2_ref[...]) + pb2_ref[...]


def _row_specs():
    return [
        pl.BlockSpec((_BLK, D), lambda i: (i, 0)),
        pl.BlockSpec((_BLK, DQ), lambda i: (i, 0)),
        pl.BlockSpec((_BLK, DQ), lambda i: (i, 0)),
        pl.BlockSpec((_BLK, DQ), lambda i: (i, 0)),
        pl.BlockSpec((_BLK, DQ), lambda i: (i, 0)),
        pl.BlockSpec((_BLK, 1), lambda i: (i, 0)),
    ]


def _w_specs():
    return [
        pl.BlockSpec((2 * D, D), lambda i: (0, 0)),
        pl.BlockSpec((1, D), lambda i: (0, 0)),
        pl.BlockSpec((D, D), lambda i: (0, 0)),
        pl.BlockSpec((1, D), lambda i: (0, 0)),
    ]


def _tc_mlp(h, aggs, dinv, lp):
    grid = N // _BLK
    return pl.pallas_call(
        _mlp_kernel,
        grid=(grid,),
        in_specs=_row_specs() + _w_specs(),
        out_specs=[
            pl.BlockSpec((_BLK, D), lambda i: (i, 0)),
            pl.BlockSpec((4, _BLK, DQ), lambda i: (0, i, 0)),
        ],
        out_shape=[
            jax.ShapeDtypeStruct((N, D), jnp.float32),
            jax.ShapeDtypeStruct((4, N, DQ), jnp.float32),
        ],
    )(h, *aggs, dinv, lp["nW1"], lp["nb1"].reshape(1, D),
      lp["nW2"], lp["nb2"].reshape(1, D))


def _tc_final(h, aggs, dinv, lp, params):
    grid = N // _BLK
    return pl.pallas_call(
        _final_kernel,
        grid=(grid,),
        in_specs=_row_specs() + _w_specs() + [
            pl.BlockSpec((D, D), lambda i: (0, 0)),
            pl.BlockSpec((1, D), lambda i: (0, 0)),
            pl.BlockSpec((D, 128), lambda i: (0, 0)),
            pl.BlockSpec((1, 128), lambda i: (0, 0)),
        ],
        out_specs=pl.BlockSpec((_BLK, 128), lambda i: (i, 0)),
        out_shape=jax.ShapeDtypeStruct((N, 128), jnp.float32),
    )(h, *aggs, dinv, lp["nW1"], lp["nb1"].reshape(1, D),
      lp["nW2"], lp["nb2"].reshape(1, D),
      params["pW1"], params["pb1"].reshape(1, D),
      params["pW2"], params["pb2"].reshape(1, 128))


def kernel(x, edge_index, edge_attr, batch, params):
    src = edge_index[0]
    dst = edge_index[1]
    pad = jnp.arange(EP - E, dtype=src.dtype)
    spad = (pad % N).reshape(-1, EKA)
    dpad = (N + pad % 8).reshape(-1, EKA)
    src2e = src.reshape(E // EKA, EKA)
    dst2e = dst.reshape(E // EKA, EKA)
    src2 = src.reshape(E // EK, EK)

    d0, d1 = _sc_degree(src2)
    h, hs, dinv = _tc_embed(x, params["emb_W"], params["emb_b"],
                            d0.reshape(NP, 1), d1.reshape(NP, 1))

    for li, lp in enumerate(params["layers"]):
        aggs = _sc_aggregate(hs.reshape(4 * N, DQ), src2e, spad, dst2e, dpad)
        if li < len(params["layers"]) - 1:
            h, hs = _tc_mlp(h, aggs, dinv, lp)
        else:
            out = _tc_final(h, aggs, dinv, lp, params)
    return out
